# baseline (speedup 1.0000x reference)
import functools

import jax
import jax.numpy as jnp
from jax import lax
from jax.experimental import pallas as pl
from jax.experimental.pallas import tpu as pltpu
from jax.experimental.pallas import tpu_sc as plsc

B = 16384
VOCAB = 10
DIM = 50
DPAD = 64
EPS = 1e-5

NC = 2
NS = 16
NW = NC * NS
BPW = B // NW
IDX_CHUNK = 128
NCHUNK = BPW // IDX_CHUNK


def _stats_body(x_ref, tbl_ref, gamma_ref, beta_ref, out_ref):
    x = x_ref[...]
    tbl = tbl_ref[...]
    counts = [jnp.sum((x == v).astype(jnp.float32)) for v in range(VOCAB)]
    inv_b = 1.0 / B
    mean = jnp.zeros((1, DIM), jnp.float32)
    for v in range(VOCAB):
        mean = mean + (counts[v] * inv_b) * tbl[v : v + 1, :]
    var = jnp.zeros((1, DIM), jnp.float32)
    for v in range(VOCAB):
        d = tbl[v : v + 1, :] - mean
        var = var + (counts[v] * inv_b) * (d * d)
    scale = gamma_ref[...] * lax.rsqrt(var + EPS)
    out_ref[...] = jnp.maximum((tbl - mean) * scale + beta_ref[...], 0.0)


_stats_call = pl.pallas_call(
    _stats_body,
    out_shape=jax.ShapeDtypeStruct((VOCAB, DIM), jnp.float32),
)


@functools.cache
def _make_gather_call():
    @functools.partial(
        pl.kernel,
        mesh=plsc.VectorSubcoreMesh(core_axis_name="c", subcore_axis_name="s"),
        out_type=jax.ShapeDtypeStruct((B, DIM), jnp.float32),
        scratch_types=[
            pltpu.VMEM((VOCAB, DIM), jnp.float32),
            pltpu.VMEM((BPW,), jnp.int32),
            pltpu.VMEM((BPW, DIM), jnp.float32),
        ],
        compiler_params=pltpu.CompilerParams(needs_layout_passes=False),
    )
    def _gather_call(tbl_hbm, idx_hbm, out_hbm, tbl_v, idx_v, out_v):
        wid = lax.axis_index("s") * NC + lax.axis_index("c")
        pltpu.sync_copy(tbl_hbm, tbl_v)
        pltpu.sync_copy(idx_hbm.at[pl.ds(wid * BPW, BPW)], idx_v)
        lanes = lax.iota(jnp.int32, 16)

        def jblk_body(jblk, carry):
            x16 = idx_v[pl.ds(jblk * 16, 16)]
            rows = jblk * 16 + lanes

            @plsc.parallel_loop(0, DIM, unroll=10)
            def d_body(d):
                dcol = jnp.full((16,), 1, jnp.int32) * d
                v = plsc.load_gather(tbl_v, [x16, dcol])
                plsc.store_scatter(out_v, [rows, dcol], v)

            return carry

        lax.fori_loop(0, BPW // 16, jblk_body, None)
        pltpu.sync_copy(out_v, out_hbm.at[pl.ds(wid * BPW, BPW)])

    return _gather_call


def kernel(x, table, gamma, beta):
    x = x.astype(jnp.int32)
    normed = _stats_call(
        x,
        table,
        gamma.reshape(1, DIM),
        beta.reshape(1, DIM),
    )
    return _make_gather_call()(normed, x)


# speedup vs baseline: 1.3439x; 1.3439x over previous
import functools

import jax
import jax.numpy as jnp
from jax import lax
from jax.experimental import pallas as pl
from jax.experimental.pallas import tpu as pltpu
from jax.experimental.pallas import tpu_sc as plsc

B = 16384
VOCAB = 10
DIM = 50
DPAD = 64
EPS = 1e-5

NC = 2
NS = 16
NW = NC * NS
BPW = B // NW
IDX_CHUNK = 128
NCHUNK = BPW // IDX_CHUNK


def _stats_body(x_ref, tbl_ref, gamma_ref, beta_ref, out_ref):
    x = x_ref[...]
    tbl = tbl_ref[...]
    counts = [jnp.sum((x == v).astype(jnp.float32)) for v in range(VOCAB)]
    inv_b = 1.0 / B
    mean = jnp.zeros((1, DIM), jnp.float32)
    for v in range(VOCAB):
        mean = mean + (counts[v] * inv_b) * tbl[v : v + 1, :]
    var = jnp.zeros((1, DIM), jnp.float32)
    for v in range(VOCAB):
        d = tbl[v : v + 1, :] - mean
        var = var + (counts[v] * inv_b) * (d * d)
    scale = gamma_ref[...] * lax.rsqrt(var + EPS)
    out_ref[...] = jnp.maximum((tbl - mean) * scale + beta_ref[...], 0.0)


_stats_call = pl.pallas_call(
    _stats_body,
    out_shape=jax.ShapeDtypeStruct((VOCAB, DIM), jnp.float32),
)


@functools.cache
def _make_gather_call():
    @functools.partial(
        pl.kernel,
        mesh=plsc.VectorSubcoreMesh(core_axis_name="c", subcore_axis_name="s"),
        out_type=jax.ShapeDtypeStruct((B, DIM), jnp.float32),
        scratch_types=[
            pltpu.VMEM((VOCAB, DIM), jnp.float32),
            pltpu.VMEM((BPW,), jnp.int32),
            pltpu.VMEM((BPW, DIM), jnp.float32),
        ],
        compiler_params=pltpu.CompilerParams(needs_layout_passes=False),
    )
    def _gather_call(tbl_hbm, idx_hbm, out_hbm, tbl_v, idx_v, out_v):
        wid = lax.axis_index("s") * NC + lax.axis_index("c")
        pltpu.sync_copy(tbl_hbm, tbl_v)
        pltpu.sync_copy(idx_hbm.at[pl.ds(wid * BPW, BPW)], idx_v)
        lanes = lax.iota(jnp.int32, 16)
        tail_m = lanes < DIM - 48

        @plsc.parallel_loop(0, BPW // 16, unroll=1)
        def blk_body(jblk):
            x16 = idx_v[pl.ds(jblk * 16, 16)]
            for k in range(16):
                xj = x16[k]
                j = jblk * 16 + k
                for d0 in (0, 16, 32):
                    out_v[j, pl.ds(d0, 16)] = tbl_v[xj, pl.ds(d0, 16)]
                dtail = 48 + lanes
                xsplat = jnp.full((16,), xj, jnp.int32)
                jsplat = jnp.full((16,), j, jnp.int32)
                v = plsc.load_gather(tbl_v, [xsplat, dtail], mask=tail_m)
                plsc.store_scatter(out_v, [jsplat, dtail], v, mask=tail_m)

        pltpu.sync_copy(out_v, out_hbm.at[pl.ds(wid * BPW, BPW)])

    return _gather_call


def kernel(x, table, gamma, beta):
    x = x.astype(jnp.int32)
    normed = _stats_call(
        x,
        table,
        gamma.reshape(1, DIM),
        beta.reshape(1, DIM),
    )
    return _make_gather_call()(normed, x)


# speedup vs baseline: 1.7411x; 1.2956x over previous
import functools

import jax
import jax.numpy as jnp
from jax import lax
from jax.experimental import pallas as pl
from jax.experimental.pallas import tpu as pltpu
from jax.experimental.pallas import tpu_sc as plsc

B = 16384
VOCAB = 10
DIM = 50
EPS = 1e-5

NC = 2
NS = 16
NW = NC * NS
BPW = B // NW
NBLK = BPW // 16


def _stats_body(x_ref, tbl_ref, gamma_ref, beta_ref, out_ref):
    x = x_ref[...]
    tbl = tbl_ref[...]
    counts = [jnp.sum((x == v).astype(jnp.float32)) for v in range(VOCAB)]
    inv_b = 1.0 / B
    mean = jnp.zeros((1, DIM), jnp.float32)
    for v in range(VOCAB):
        mean = mean + (counts[v] * inv_b) * tbl[v : v + 1, :]
    var = jnp.zeros((1, DIM), jnp.float32)
    for v in range(VOCAB):
        d = tbl[v : v + 1, :] - mean
        var = var + (counts[v] * inv_b) * (d * d)
    scale = gamma_ref[...] * lax.rsqrt(var + EPS)
    normed = jnp.maximum((tbl - mean) * scale + beta_ref[...], 0.0)
    out_ref[:, :VOCAB] = normed.T
    out_ref[:, VOCAB:] = jnp.zeros((DIM, 16 - VOCAB), jnp.float32)


_stats_call = pl.pallas_call(
    _stats_body,
    out_shape=jax.ShapeDtypeStruct((DIM, 16), jnp.float32),
)


@functools.cache
def _make_gather_call():
    @functools.partial(
        pl.kernel,
        mesh=plsc.VectorSubcoreMesh(core_axis_name="c", subcore_axis_name="s"),
        out_type=jax.ShapeDtypeStruct((DIM, B), jnp.float32),
        scratch_types=[
            pltpu.VMEM((DIM, 16), jnp.float32),
            pltpu.VMEM((BPW,), jnp.int32),
            pltpu.VMEM((DIM, BPW), jnp.float32),
        ],
        compiler_params=pltpu.CompilerParams(needs_layout_passes=False),
    )
    def _gather_call(tbl_hbm, idx_hbm, out_hbm, tbl_v, idx_v, out_v):
        wid = lax.axis_index("s") * NC + lax.axis_index("c")
        pltpu.sync_copy(tbl_hbm, tbl_v)
        pltpu.sync_copy(idx_hbm.at[pl.ds(wid * BPW, BPW)], idx_v)
        cols = [tbl_v[d, :] for d in range(DIM)]
        dnums = lax.GatherDimensionNumbers(
            offset_dims=(), collapsed_slice_dims=(0,), start_index_map=(0,)
        )
        for jblk in range(NBLK):
            x16 = idx_v[pl.ds(jblk * 16, 16)]
            xi = x16[:, None]
            for d in range(DIM):
                out_v[d, pl.ds(jblk * 16, 16)] = lax.gather(
                    cols[d],
                    xi,
                    dnums,
                    (1,),
                    mode=lax.GatherScatterMode.PROMISE_IN_BOUNDS,
                )
        pltpu.sync_copy(out_v, out_hbm.at[:, pl.ds(wid * BPW, BPW)])

    return _gather_call


def kernel(x, table, gamma, beta):
    x = x.astype(jnp.int32)
    tbl_t = _stats_call(
        x,
        table,
        gamma.reshape(1, DIM),
        beta.reshape(1, DIM),
    )
    return _make_gather_call()(tbl_t, x).T
